# baseline (speedup 1.0000x reference)
import functools

import jax
import jax.numpy as jnp
from jax import lax
from jax.experimental import pallas as pl
from jax.experimental.pallas import tpu as pltpu
from jax.experimental.pallas import tpu_sc as plsc

_B = 16384
_F = 26
_E = 16
_VPF = 40000
_V = _F * _VPF
_NC = 2
_NS = 16
_NW = _NC * _NS
_L = 16

_NTILES = _V // 128
_GT = 4
_NGROUPS = _NTILES // _GT
_K1_ITERS = -(-_NGROUPS // _NW)
_SLAB_ROWS = _V // 8

_BPW = _B // _NW
_NQ = 4
_BPQ = _BPW // _NQ
_QIVECS = _BPQ // _L


_CB = 16000
_RB = _CB // 8


@functools.lru_cache(maxsize=1)
def _build_kernels():
    mesh = plsc.VectorSubcoreMesh(core_axis_name="c", subcore_axis_name="s")
    params = pltpu.CompilerParams(needs_layout_passes=False)

    def untile_body(in_ref, out_ref):
        t3 = in_ref[...].T.reshape(_RB, 8, _E)
        for m in range(8):
            out_ref[:, pl.ds(m * _E, _E)] = t3[:, m, :]

    tc_untile = pl.pallas_call(
        untile_body,
        grid=(_V // _CB,),
        in_specs=[pl.BlockSpec((_E, _CB), lambda i: (0, i))],
        out_specs=pl.BlockSpec((_RB, 128), lambda i: (i, 0)),
        out_shape=jax.ShapeDtypeStruct((_SLAB_ROWS, 128), jnp.float32),
    )

    @functools.partial(
        pl.kernel,
        mesh=mesh,
        out_type=jax.ShapeDtypeStruct((_F, _E, _B), jnp.float32),
        scratch_types=[
            pltpu.VMEM((_F, _BPW), jnp.int32),
            [pltpu.VMEM((_BPQ,), jnp.int32) for _ in range(_NQ)],
            [pltpu.VMEM((_BPQ,), jnp.int32) for _ in range(_NQ)],
            [pltpu.VMEM((_BPQ, 128), jnp.float32) for _ in range(_NQ)],
            [pltpu.VMEM((_E, _BPQ), jnp.float32) for _ in range(_NQ)],
            [pltpu.SemaphoreType.DMA for _ in range(_NQ)],
            [pltpu.SemaphoreType.DMA for _ in range(_NQ)],
        ],
        compiler_params=params,
    )
    def gather_kernel(xt_hbm, slab_hbm, out_hbm, xall, gidxs, cbases,
                      gbufs, obufs, gsems, vsems):
        w = lax.axis_index("s") * _NC + lax.axis_index("c")
        b0 = w * _BPW
        iota = lax.iota(jnp.int32, _L)

        pltpu.sync_copy(xt_hbm.at[:, pl.ds(b0, _BPW)], xall)

        def start_gather(f, q):
            @pl.when(f < _F)
            def _():
                off = f * _VPF

                def idx_body(j, carry):
                    sl = pl.ds(q * _BPQ + j * _L, _L)
                    dl = pl.ds(j * _L, _L)
                    u = xall[f, sl] + off
                    cbases[q][dl] = (u & 7) * _L
                    gidxs[q][dl] = u >> 3
                    return carry

                lax.fori_loop(0, _QIVECS, idx_body, 0, unroll=4)
                pltpu.async_copy(slab_hbm.at[gidxs[q]], gbufs[q], gsems[q])

        for q in range(_NQ):
            start_gather(0, q)

        def field_body(f, carry):
            for q in range(_NQ):
                pltpu.make_async_copy(
                    slab_hbm.at[gidxs[q]], gbufs[q], gsems[q]
                ).wait()

                @pl.when(f >= 1)
                def _():
                    pltpu.make_async_copy(
                        obufs[q], out_hbm.at[0, :, pl.ds(0, _BPQ)], vsems[q]
                    ).wait()

                def trans_body(j, carry2):
                    rows = j * _L + iota
                    cvec = cbases[q][pl.ds(j * _L, _L)]
                    for e in range(_E):
                        vec = plsc.load_gather(gbufs[q], [rows, cvec + e])
                        obufs[q][e, pl.ds(j * _L, _L)] = vec
                    return carry2

                lax.fori_loop(0, _QIVECS, trans_body, 0)

                pltpu.async_copy(
                    obufs[q],
                    out_hbm.at[f, :, pl.ds(b0 + q * _BPQ, _BPQ)],
                    vsems[q],
                )
                start_gather(f + 1, q)
            return carry

        lax.fori_loop(0, _F, field_body, 0)

        for q in range(_NQ):
            pltpu.make_async_copy(
                obufs[q], out_hbm.at[0, :, pl.ds(0, _BPQ)], vsems[q]
            ).wait()

    return tc_untile, gather_kernel


def kernel(x, table):
    untile_kernel, gather_kernel = _build_kernels()
    x_t = x.T
    tab_t = table.T
    slab = untile_kernel(tab_t)
    out_t = gather_kernel(x_t, slab)
    return jnp.transpose(out_t, (2, 0, 1))


# speedup vs baseline: 1.6413x; 1.6413x over previous
import functools

import jax
import jax.numpy as jnp
from jax import lax
from jax.experimental import pallas as pl
from jax.experimental.pallas import tpu as pltpu
from jax.experimental.pallas import tpu_sc as plsc

_B = 16384
_F = 26
_E = 16
_VPF = 40000
_V = _F * _VPF
_NC = 2
_NS = 16
_NW = _NC * _NS
_L = 16

_F_TC = 10
_V_TC = _F_TC * _VPF
_TILE0_SC = _V_TC // 128
_ROW0_SC = _V_TC // 8

_CB = 16000
_RB = _CB // 8

_GT = 4
_NGROUPS = (_V - _V_TC) // 128 // _GT
_K1_ITERS = -(-_NGROUPS // _NW)

_BPW = _B // _NW
_NQ = 4
_BPQ = _BPW // _NQ
_QIVECS = _BPQ // _L


@functools.lru_cache(maxsize=1)
def _build_kernels():
    mesh = plsc.VectorSubcoreMesh(core_axis_name="c", subcore_axis_name="s")
    params = pltpu.CompilerParams(needs_layout_passes=False)

    def tc_untile_body(in_ref, out_ref):
        t3 = in_ref[...].T.reshape(_RB, 8, _E)
        for m in range(8):
            out_ref[:, pl.ds(m * _E, _E)] = t3[:, m, :]

    tc_untile = pl.pallas_call(
        tc_untile_body,
        grid=(_V_TC // _CB,),
        in_specs=[pl.BlockSpec((_E, _CB), lambda i: (0, i))],
        out_specs=pl.BlockSpec((_RB, 128), lambda i: (i, 0)),
        out_shape=jax.ShapeDtypeStruct((_ROW0_SC, 128), jnp.float32),
    )

    @functools.partial(
        pl.kernel,
        mesh=mesh,
        out_type=jax.ShapeDtypeStruct(((_V - _V_TC) // 8, 128), jnp.float32),
        scratch_types=[
            pltpu.VMEM((_E, _GT * 128), jnp.float32),
            pltpu.VMEM((_E, _GT * 128), jnp.float32),
            pltpu.VMEM((_GT * _E, 128), jnp.float32),
            pltpu.VMEM((_GT * _E, 128), jnp.float32),
            pltpu.SemaphoreType.DMA,
            pltpu.SemaphoreType.DMA,
            pltpu.SemaphoreType.DMA,
            pltpu.SemaphoreType.DMA,
        ],
        compiler_params=params,
    )
    def sc_untile(tabt_hbm, slab_hbm, tb0, tb1, rb0, rb1, is0, is1, os0, os1):
        w = lax.axis_index("s") * _NC + lax.axis_index("c")
        iota = lax.iota(jnp.int32, _L)
        tbufs, rbufs, isems, osems = (tb0, tb1), (rb0, rb1), (is0, is1), (os0, os1)

        rows_tc = [[((jnp.full((_L,), 16 * c, jnp.int32) + iota) >> 3) + 16 * t
                    for c in range(8)] for t in range(_GT)]
        colb_c = [((jnp.full((_L,), 16 * c, jnp.int32) + iota) & 7) * _L
                  for c in range(8)]

        def start_in(k, par):
            g = w + k * _NW
            @pl.when(g < _NGROUPS)
            def _():
                jt = _TILE0_SC + g * _GT
                pltpu.async_copy(
                    tabt_hbm.at[:, pl.ds(jt * 128, _GT * 128)],
                    tbufs[par], isems[par],
                )

        start_in(0, 0)
        start_in(1, 1)

        def group_pair(kk, carry):
            for par in range(2):
                k = kk * 2 + par
                g = w + k * _NW

                @pl.when(g < _NGROUPS)
                def _():
                    @pl.when(k >= 2)
                    def _():
                        pltpu.make_async_copy(
                            rbufs[par], slab_hbm.at[pl.ds(0, _GT * _E), :],
                            osems[par],
                        ).wait()
                    pltpu.make_async_copy(
                        tabt_hbm.at[:, pl.ds(0, _GT * 128)], tbufs[par],
                        isems[par],
                    ).wait()
                    for t in range(_GT):
                        for c in range(8):
                            for e in range(_E):
                                vec = tbufs[par][e, pl.ds(t * 128 + c * _L, _L)]
                                plsc.store_scatter(
                                    rbufs[par], [rows_tc[t][c], colb_c[c] + e],
                                    vec,
                                )
                    pltpu.async_copy(
                        rbufs[par],
                        slab_hbm.at[pl.ds(g * _GT * _E, _GT * _E), :],
                        osems[par],
                    )
                    start_in(k + 2, par)
            return carry

        lax.fori_loop(0, _K1_ITERS // 2, group_pair, 0)

        for par in range(2):
            pltpu.make_async_copy(
                rbufs[par], slab_hbm.at[pl.ds(0, _GT * _E), :], osems[par]
            ).wait()

    @functools.partial(
        pl.kernel,
        mesh=mesh,
        out_type=jax.ShapeDtypeStruct((_F, _E, _B), jnp.float32),
        scratch_types=[
            pltpu.VMEM((_F, _BPW), jnp.int32),
            [pltpu.VMEM((_BPQ,), jnp.int32) for _ in range(_NQ)],
            [pltpu.VMEM((_BPQ,), jnp.int32) for _ in range(_NQ)],
            [pltpu.VMEM((_BPQ, 128), jnp.float32) for _ in range(_NQ)],
            [pltpu.VMEM((_E, _BPQ), jnp.float32) for _ in range(_NQ)],
            [pltpu.SemaphoreType.DMA for _ in range(_NQ)],
            [pltpu.SemaphoreType.DMA for _ in range(_NQ)],
        ],
        compiler_params=params,
    )
    def gather_kernel(xt_hbm, slab_tc_hbm, slab_sc_hbm, out_hbm, xall, gidxs,
                      cbases, gbufs, obufs, gsems, vsems):
        w = lax.axis_index("s") * _NC + lax.axis_index("c")
        b0 = w * _BPW
        iota = lax.iota(jnp.int32, _L)

        pltpu.sync_copy(xt_hbm.at[:, pl.ds(b0, _BPW)], xall)

        def next_field(f):
            return lax.select(
                f == _F - 1, jnp.int32(0),
                lax.select(f == _F_TC - 1, jnp.int32(_F), f + 1),
            )

        def start_gather(f, q):
            def prep(row_off):
                def idx_body(j, carry):
                    sl = pl.ds(q * _BPQ + j * _L, _L)
                    dl = pl.ds(j * _L, _L)
                    u = xall[f, sl] + f * _VPF
                    cbases[q][dl] = (u & 7) * _L
                    gidxs[q][dl] = (u >> 3) - row_off
                    return carry

                lax.fori_loop(0, _QIVECS, idx_body, 0, unroll=4)

            @pl.when(f < _F_TC)
            def _():
                prep(0)
                pltpu.async_copy(slab_tc_hbm.at[gidxs[q]], gbufs[q], gsems[q])

            @pl.when((f >= _F_TC) & (f < _F))
            def _():
                prep(_ROW0_SC)
                pltpu.async_copy(slab_sc_hbm.at[gidxs[q]], gbufs[q], gsems[q])

        for q in range(_NQ):
            start_gather(jnp.int32(_F_TC), q)

        def body(i, carry):
            f = lax.select(i < _F - _F_TC, i + _F_TC, i - (_F - _F_TC))
            for q in range(_NQ):
                pltpu.make_async_copy(
                    slab_sc_hbm.at[gidxs[q]], gbufs[q], gsems[q]
                ).wait()

                @pl.when(carry >= 1)
                def _():
                    pltpu.make_async_copy(
                        obufs[q], out_hbm.at[0, :, pl.ds(0, _BPQ)], vsems[q]
                    ).wait()

                def trans_body(j, carry2):
                    rows = j * _L + iota
                    cvec = cbases[q][pl.ds(j * _L, _L)]
                    for e in range(_E):
                        vec = plsc.load_gather(gbufs[q], [rows, cvec + e])
                        obufs[q][e, pl.ds(j * _L, _L)] = vec
                    return carry2

                lax.fori_loop(0, _QIVECS, trans_body, 0)

                pltpu.async_copy(
                    obufs[q],
                    out_hbm.at[f, :, pl.ds(b0 + q * _BPQ, _BPQ)],
                    vsems[q],
                )
                start_gather(next_field(f), q)
            return carry + 1

        lax.fori_loop(0, _F, body, 0)

        for q in range(_NQ):
            pltpu.make_async_copy(
                obufs[q], out_hbm.at[0, :, pl.ds(0, _BPQ)], vsems[q]
            ).wait()

    return tc_untile, sc_untile, gather_kernel


def kernel(x, table):
    tc_untile, sc_untile, gather_kernel = _build_kernels()
    x_t = x.T
    tab_t = table.T
    slab_tc = tc_untile(tab_t)
    slab_sc = sc_untile(tab_t)
    out_t = gather_kernel(x_t, slab_tc, slab_sc)
    return jnp.transpose(out_t, (2, 0, 1))
